# baseline (speedup 1.0000x reference)
import functools

import jax
import jax.numpy as jnp
from jax import lax
from jax.experimental import pallas as pl
from jax.experimental.pallas import tpu as pltpu
from jax.experimental.pallas import tpu_sc as plsc

N_U = 30000
N_I = 20000
E = 800000
D = 64
L = 2
B = 4096
N = N_U + N_I
ALPHA = 0.2
INV_TEMP = 2.0
LAM1 = 0.2
LAM2 = 0.2
REG = 1e-5

_SC_TILES = 16
_STREAM = 128
_SCHUNK = 2
_HD = D // 2
_TSTREAMS = -(-E // (_SC_TILES * _STREAM))
_TSTREAMS = -(-_TSTREAMS // _SCHUNK) * _SCHUNK
_EP = _SC_TILES * _TSTREAMS * _STREAM
_NSC = _TSTREAMS // _SCHUNK
_NSUPT = _SC_TILES * _NSC
_ACC_ROWS = 51200
_IBASE = 30720
_DUMMY = N_U
_DUMMY_I = _IBASE + N_I
_ZROWS = 128


def _sc_mesh():
    return plsc.VectorSubcoreMesh(core_axis_name="c", subcore_axis_name="s")


_BCAST_DN = lax.GatherDimensionNumbers(
    offset_dims=(), collapsed_slice_dims=(0,), start_index_map=(0,))


def _bcast16(v16, t):
    idx = jnp.full((16, 1), t, jnp.int32)
    return lax.gather(v16, idx, _BCAST_DN, (1,),
                      mode=lax.GatherScatterMode.PROMISE_IN_BOUNDS)


def _spmm_body(scaled, table, pack_u, pack_i, out_u, out_i, idx_v, bufs, zbuf,
               acc, gsem, ssem):
    c = lax.axis_index("c")
    s = lax.axis_index("s")

    for r in range(_ZROWS):
        for q in range(_HD // 16):
            zbuf[r, pl.ds(q * 16, 16)] = jnp.zeros((16,), jnp.float32)
    rows_per_tile = _ACC_ROWS // _SC_TILES
    for t in range(rows_per_tile // _ZROWS):
        pltpu.sync_copy(zbuf,
                        acc.at[pl.ds(s * rows_per_tile + t * _ZROWS, _ZROWS)])
    plsc.subcore_barrier()

    def one_pass(pack):
        def fire(k, par):
            sup = s * _NSC + k
            pltpu.sync_copy(pack.at[c, sup], idx_v.at[par])
            for j in range(_SCHUNK):
                pltpu.async_copy(table.at[idx_v.at[par, 0, j]],
                                 bufs.at[par, j], gsem.at[par])

        def process(par):
            for j in range(_SCHUNK):
                pltpu.make_async_copy(table.at[idx_v.at[par, 0, j]],
                                      bufs.at[par, j], gsem.at[par]).wait()
                if scaled:
                    for g16 in range(_STREAM // 16):
                        v16 = plsc.bitcast(
                            idx_v[par, 2, j, pl.ds(g16 * 16, 16)], jnp.float32)
                        for t in range(16):
                            r = g16 * 16 + t
                            bc = _bcast16(v16, t)
                            for q in range(_HD // 16):
                                bufs[par, j, r, pl.ds(q * 16, 16)] = (
                                    bufs[par, j, r, pl.ds(q * 16, 16)] * bc)
                pltpu.async_copy(bufs.at[par, j], acc.at[idx_v.at[par, 1, j]],
                                 ssem.at[par], add=True)

        def drain_scatter(par):
            for j in range(_SCHUNK):
                pltpu.make_async_copy(bufs.at[par, j],
                                      acc.at[idx_v.at[par, 1, j]],
                                      ssem.at[par]).wait()

        fire(0, 0)

        def step(k, carry):
            par = k % 2

            @pl.when(k >= 1)
            def _():
                drain_scatter(1 - par)

            @pl.when(k + 1 < _NSC)
            def _():
                fire(k + 1, 1 - par)

            process(par)
            return carry

        lax.fori_loop(0, _NSC, step, 0, unroll=False)
        drain_scatter((_NSC - 1) % 2)

    one_pass(pack_u)
    one_pass(pack_i)
    plsc.subcore_barrier()

    def writeout(base, n_out, out):
        rows_per = (n_out // _SC_TILES) // 8 * 8
        tail = n_out - 15 * rows_per

        @pl.when(s < 15)
        def _():
            o = pl.multiple_of(s * rows_per, 8)
            pltpu.sync_copy(acc.at[pl.ds(base + o, rows_per)],
                            out.at[c, pl.ds(o, rows_per), :])

        @pl.when(s == 15)
        def _():
            pltpu.sync_copy(acc.at[pl.ds(base + 15 * rows_per, tail)],
                            out.at[c, pl.ds(15 * rows_per, tail), :])

    writeout(0, N_U, out_u)
    writeout(_IBASE, N_I, out_i)


def _make_spmm(scaled):
    body = functools.partial(_spmm_body, scaled)
    kern = pl.kernel(
        body,
        out_type=[jax.ShapeDtypeStruct((2, N_U, _HD), jnp.float32),
                  jax.ShapeDtypeStruct((2, N_I, _HD), jnp.float32)],
        mesh=_sc_mesh(),
        scratch_types=[
            pltpu.VMEM((2, 3, _SCHUNK, _STREAM), jnp.int32),
            pltpu.VMEM((2, _SCHUNK, _STREAM, _HD), jnp.float32),
            pltpu.VMEM((_ZROWS, _HD), jnp.float32),
            pltpu.VMEM_SHARED((_ACC_ROWS, _HD), jnp.float32),
            pltpu.SemaphoreType.DMA((2,)),
            pltpu.SemaphoreType.DMA((2,)),
        ],
        compiler_params=pltpu.CompilerParams(use_tc_tiling_on_sc=False,
                                             needs_layout_passes=False),
    )
    return kern


_GTASKS = (
    (0, 0), (0, 1), (3, 0), (3, 1),
    (1, 0), (1, 1), (4, 0), (4, 1),
    (2, 0), (2, 1), (5, 0), (5, 1),
    (6, 0), (6, 1), (6, 2),
)
_NG = len(_GTASKS)


def _gather_body(e3_0, e1_0, e2_0, e3_1, e1_1, e2_1, latsf, idx3, rawidx,
                 g_out, mask_out, idx_v, buf, idxfull, tbl, mbuf, sem):
    c = lax.axis_index("c")
    s = lax.axis_index("s")
    w = s * 2 + c
    tables = (e3_0, e1_0, e2_0, e3_1, e1_1, e2_1, latsf)

    for p in range(3):
        pltpu.sync_copy(idx3.at[pl.ds(p * B + w * _STREAM, _STREAM)],
                        idx_v.at[p])

    def desc(t):
        tb, p = _GTASKS[t]
        return pltpu.make_async_copy(tables[tb].at[idx_v.at[p]], buf.at[t % 4],
                                     sem)

    for t in range(_NG):
        desc(t).start()
        if t >= 3:
            desc(t - 3).wait()
            pltpu.sync_copy(buf.at[(t - 3) % 4],
                            g_out.at[t - 3, pl.ds(w * _STREAM, _STREAM), :])
    for t in range(_NG - 3, _NG):
        desc(t).wait()
        pltpu.sync_copy(buf.at[t % 4],
                        g_out.at[t, pl.ds(w * _STREAM, _STREAM), :])

    @pl.when(s == 0)
    def _():
        co = pl.multiple_of(c * B, 8)
        pltpu.sync_copy(rawidx.at[pl.ds(co, B)], idxfull)
        for gq in range(B // 16):
            i16 = idxfull[pl.ds(gq * 16, 16)]
            p16 = lax.iota(jnp.int32, 16) + jnp.full((16,), gq * 16, jnp.int32)
            plsc.store_scatter(tbl, [i16], p16)
        for gq in range(B // 16):
            i16 = idxfull[pl.ds(gq * 16, 16)]
            p16 = lax.iota(jnp.int32, 16) + jnp.full((16,), gq * 16, jnp.int32)
            rep = plsc.load_gather(tbl, [i16])
            mbuf[pl.ds(gq * 16, 16)] = (rep == p16).astype(jnp.float32)
        pltpu.sync_copy(mbuf, mask_out.at[pl.ds(co, B)])


def _make_gather():
    return pl.kernel(
        _gather_body,
        out_type=[jax.ShapeDtypeStruct((_NG, B, D), jnp.float32),
                  jax.ShapeDtypeStruct((2 * B,), jnp.float32)],
        mesh=_sc_mesh(),
        scratch_types=[
            pltpu.VMEM((3, _STREAM), jnp.int32),
            pltpu.VMEM((4, _STREAM, D), jnp.float32),
            pltpu.VMEM((B,), jnp.int32),
            pltpu.VMEM((N_U,), jnp.int32),
            pltpu.VMEM((B,), jnp.float32),
            pltpu.SemaphoreType.DMA,
        ],
        compiler_params=pltpu.CompilerParams(use_tc_tiling_on_sc=False,
                                             needs_layout_passes=False),
    )


_BLK = 2000
_NBLK = N // _BLK
_NBLK_U = N_U // _BLK


def _stats0_kernel(e_ref, h_ref, out_ref):
    i = pl.program_id(0)

    @pl.when(i == 0)
    def _():
        out_ref[...] = jnp.zeros_like(out_ref)

    e = e_ref[...]
    h = h_ref[...]
    out_ref[0, :] += jnp.sum(e, axis=0)
    out_ref[1, :] += jnp.sum(e * e, axis=0)
    out_ref[2, :] += jnp.sum(h * h, axis=0)


def _tc_stats0(embeds, hyp):
    return pl.pallas_call(
        _stats0_kernel,
        grid=(_NBLK,),
        in_specs=[pl.BlockSpec((_BLK, D), lambda i: (i, 0)),
                  pl.BlockSpec((_BLK, D), lambda i: (i, 0))],
        out_specs=pl.BlockSpec((8, D), lambda i: (0, 0)),
        out_shape=jax.ShapeDtypeStruct((8, D), jnp.float32),
        compiler_params=pltpu.CompilerParams(dimension_semantics=("arbitrary",)),
    )(embeds, hyp)


def _bn_kernel(lats_ref, huu_ref, par_ref, bn_ref, g_ref):
    i = pl.program_id(0)
    bn = lats_ref[...] * par_ref[0, :] + par_ref[1, :]
    bn_ref[0] = bn[:, :_HD]
    bn_ref[1] = bn[:, _HD:]

    @pl.when((i == 0) | (i == _NBLK_U))
    def _():
        g_ref[...] = jnp.zeros_like(g_ref)

    g_ref[0] += lax.dot_general(huu_ref[...], bn, (((0,), (0,)), ((), ())),
                                preferred_element_type=jnp.float32)


def _tc_bn(lats, huu, par):
    return pl.pallas_call(
        _bn_kernel,
        grid=(_NBLK,),
        in_specs=[pl.BlockSpec((_BLK, D), lambda i: (i, 0)),
                  pl.BlockSpec((_BLK, D), lambda i: (i, 0)),
                  pl.BlockSpec((8, D), lambda i: (0, 0))],
        out_specs=[pl.BlockSpec((2, _BLK, _HD), lambda i: (0, i, 0)),
                   pl.BlockSpec((1, D, D),
                                lambda i: (jnp.where(i < _NBLK_U, 0, 1), 0, 0))],
        out_shape=[jax.ShapeDtypeStruct((2, N, _HD), jnp.float32),
                   jax.ShapeDtypeStruct((2, D, D), jnp.float32)],
        compiler_params=pltpu.CompilerParams(dimension_semantics=("arbitrary",)),
    )(lats, huu, par)


def _combine_kernel(lats_ref, e1_ref, huu_ref, g_ref, e2_ref, e3_ref, nl_ref,
                    st_ref):
    i = pl.program_id(0)
    e2 = lax.dot_general(huu_ref[...], g_ref[0], (((1,), (0,)), ((), ())),
                         preferred_element_type=jnp.float32)
    e3 = e1_ref[...] + lats_ref[...]
    nrm = jnp.sqrt(jnp.sum(e2 * e2, axis=1, keepdims=True))
    nl = e3 + ALPHA * e2 / jnp.maximum(nrm, 1e-12)
    e2_ref[...] = e2
    e3_ref[...] = e3
    nl_ref[...] = nl

    @pl.when(i == 0)
    def _():
        st_ref[...] = jnp.zeros_like(st_ref)

    st_ref[0, :] += jnp.sum(nl, axis=0)
    st_ref[1, :] += jnp.sum(nl * nl, axis=0)


def _tc_combine(lats, e1, huu, G):
    return pl.pallas_call(
        _combine_kernel,
        grid=(_NBLK,),
        in_specs=[pl.BlockSpec((_BLK, D), lambda i: (i, 0)),
                  pl.BlockSpec((_BLK, D), lambda i: (i, 0)),
                  pl.BlockSpec((_BLK, D), lambda i: (i, 0)),
                  pl.BlockSpec((1, D, D),
                               lambda i: (jnp.where(i < _NBLK_U, 0, 1), 0, 0))],
        out_specs=[pl.BlockSpec((_BLK, D), lambda i: (i, 0)),
                   pl.BlockSpec((_BLK, D), lambda i: (i, 0)),
                   pl.BlockSpec((_BLK, D), lambda i: (i, 0)),
                   pl.BlockSpec((8, D), lambda i: (0, 0))],
        out_shape=[jax.ShapeDtypeStruct((N, D), jnp.float32),
                   jax.ShapeDtypeStruct((N, D), jnp.float32),
                   jax.ShapeDtypeStruct((N, D), jnp.float32),
                   jax.ShapeDtypeStruct((8, D), jnp.float32)],
        compiler_params=pltpu.CompilerParams(dimension_semantics=("arbitrary",)),
    )(lats, e1, huu, G)


_RB = 256
_NRB = B // _RB
_P2SLOT = [4, 5, 6, 7, 8, 9, 10, 11]


def _contrast_kernel(p1_ref, p2_ref, p2d_ref, mc_ref, mr_ref, out_ref):
    rb = pl.program_id(1)
    p2 = p2_ref[0]
    p2 = p2 / jnp.maximum(
        jnp.sqrt(jnp.sum(p2 * p2, axis=1, keepdims=True)), 1e-12)
    p1 = p1_ref[0]
    p1 = p1 / jnp.maximum(
        jnp.sqrt(jnp.sum(p1 * p1, axis=1, keepdims=True)), 1e-12)
    p1 = p1 * INV_TEMP
    s = lax.dot_general(p1.astype(jnp.bfloat16), p2.astype(jnp.bfloat16),
                        (((1,), (1,)), ((), ())),
                        preferred_element_type=jnp.float32)
    mc = mc_ref[0, 0, :]
    ex = jnp.exp(s) * mc[None, :]
    lse = jnp.log(jnp.sum(ex, axis=1))
    p2d = p2d_ref[0]
    p2d = p2d / jnp.maximum(
        jnp.sqrt(jnp.sum(p2d * p2d, axis=1, keepdims=True)), 1e-12)
    diag = jnp.sum(p1 * p2d, axis=1)
    mr = mr_ref[0, 0, :]
    contrib = jnp.where(mr > 0.0, diag - lse, 0.0)
    part = jnp.sum(contrib.reshape(_RB // 128, 128), axis=0)

    @pl.when(rb == 0)
    def _():
        out_ref[...] = jnp.zeros_like(out_ref)

    out_ref[0, 0, :] += part


def _tc_contrast(g, masks):
    return pl.pallas_call(
        _contrast_kernel,
        grid=(8, _NRB),
        in_specs=[
            pl.BlockSpec((1, _RB, D), lambda c, rb: (c, rb, 0)),
            pl.BlockSpec((1, B, D), lambda c, rb: (c + 4, 0, 0)),
            pl.BlockSpec((1, _RB, D), lambda c, rb: (c + 4, rb, 0)),
            pl.BlockSpec((1, 1, B), lambda c, rb: (c % 2, 0, 0)),
            pl.BlockSpec((1, 1, _RB), lambda c, rb: (c % 2, 0, rb)),
        ],
        out_specs=pl.BlockSpec((1, 8, 128), lambda c, rb: (c, 0, 0)),
        out_shape=jax.ShapeDtypeStruct((8, 8, 128), jnp.float32),
        compiler_params=pltpu.CompilerParams(
            dimension_semantics=("arbitrary", "arbitrary")),
    )(g, g, g, masks, masks)


def _bpr_kernel(u_ref, p_ref, n_ref, out_ref):
    u = u_ref[0]
    ps = jnp.sum(u * p_ref[0], axis=1)
    ns = jnp.sum(u * n_ref[0], axis=1)
    x = (ps - ns) * (1.0 / D)
    prob = 1.0 / (1.0 + jnp.exp(-x))
    lg = jnp.log(prob + 1e-15)
    part = jnp.sum(lg.reshape(B // 128, 128), axis=0)
    row = lax.broadcasted_iota(jnp.int32, (8, 128), 0)
    out_ref[...] = jnp.where(row == 0, part[None, :], 0.0)


def _tc_bpr(g):
    return pl.pallas_call(
        _bpr_kernel,
        grid=(1,),
        in_specs=[pl.BlockSpec((1, B, D), lambda i: (12, 0, 0)),
                  pl.BlockSpec((1, B, D), lambda i: (13, 0, 0)),
                  pl.BlockSpec((1, B, D), lambda i: (14, 0, 0))],
        out_specs=pl.BlockSpec((8, 128), lambda i: (0, 0)),
        out_shape=jax.ShapeDtypeStruct((8, 128), jnp.float32),
    )(g, g, g)


def _bn_params(colsum, colsumsq, gamma, beta):
    m = colsum / N
    v = colsumsq / N - m * m
    scale = gamma * lax.rsqrt(v + 1e-5)
    shift = beta - m * scale
    return jnp.concatenate(
        [scale[None, :], shift[None, :], jnp.zeros((6, D), jnp.float32)], axis=0)


def kernel(uids, iids, pos, neg, adj_rows, adj_cols, adj_vals, adj_norm_vals,
           u_embeds, i_embeds, u_hyper_graph, i_hyper_graph, bn_gamma, bn_beta):
    del iids, adj_vals
    embeds = jnp.concatenate([u_embeds, i_embeds], axis=0)
    hyp = jnp.concatenate([u_hyper_graph, i_hyper_graph], axis=0)

    pad = _EP - E
    shp = (_NSUPT, _SCHUNK, _STREAM)
    scat_u = jnp.pad(adj_rows, (0, pad), constant_values=_DUMMY).reshape(shp)
    scat_i = jnp.pad(adj_cols + _IBASE, (0, pad),
                     constant_values=_DUMMY_I).reshape(shp)
    gcols_p = jnp.pad(adj_cols + N_U, (0, pad), constant_values=0).reshape(shp)
    grows_p = jnp.pad(adj_rows, (0, pad), constant_values=0).reshape(shp)
    vals_p = lax.bitcast_convert_type(
        jnp.pad(adj_norm_vals, (0, pad)).reshape(shp), jnp.int32)

    def mkpack(gp, sp):
        return jnp.stack([jnp.stack([gp, sp, vals_p], axis=1),
                          jnp.stack([gp + N, sp, vals_p], axis=1)])

    pack_u = mkpack(gcols_p, scat_u)
    pack_i = mkpack(grows_p, scat_i)

    spmm_plain = _make_spmm(False)
    spmm_scaled = _make_spmm(True)

    def split2(x):
        return jnp.concatenate([x[:, :_HD], x[:, _HD:]], axis=0)

    def join2(u2, i2):
        return jnp.concatenate(
            [jnp.concatenate([u2[0], u2[1]], axis=1),
             jnp.concatenate([i2[0], i2[1]], axis=1)], axis=0)

    hyp2 = split2(hyp)
    huu = join2(*spmm_plain(hyp2, pack_u, pack_i))
    stats0 = _tc_stats0(embeds, hyp)

    lats = embeds
    colsum, colsumsq = stats0[0], stats0[1]
    e1s, e2s, e3s = [], [], []
    for l in range(L):
        par = _bn_params(colsum, colsumsq, bn_gamma[l], bn_beta[l])
        bn2, G = _tc_bn(lats, huu, par)
        bn2f = bn2.reshape(2 * N, _HD)
        e1 = join2(*spmm_scaled(bn2f, pack_u, pack_i))
        e2, e3, nlats, st = _tc_combine(lats, e1, huu, G)
        e1s.append(e1)
        e2s.append(e2)
        e3s.append(e3)
        lats = nlats
        colsum, colsumsq = st[0], st[1]

    idx3 = jnp.concatenate([uids, pos + N_U, neg + N_U])
    rawidx = jnp.concatenate([uids, pos])
    g, masks = _make_gather()(e3s[0], e1s[0], e2s[0], e3s[1], e1s[1], e2s[1],
                              lats, idx3, rawidx)

    lsum = _tc_contrast(g, masks.reshape(2, 1, B))
    bpr = _tc_bpr(g)

    cnt = jnp.sum(masks.reshape(2, B), axis=1)
    d_c = jnp.sum(lsum, axis=(1, 2))
    loss_c = -d_c / jnp.where(jnp.arange(8) % 2 == 0, cnt[0], cnt[1])
    loss_s = jnp.sum(loss_c[0:4])
    loss_s2 = jnp.sum(loss_c[4:8])
    loss_s_tot = (loss_s * LAM1 + loss_s2 * LAM2) / L

    loss_r = -jnp.sum(bpr) / B

    loss_reg = REG * (jnp.sum(stats0[1]) + jnp.sum(stats0[2])
                      + jnp.sum(bn_gamma ** 2) + jnp.sum(bn_beta ** 2))

    loss = loss_r + loss_s_tot + loss_reg
    return loss, loss_r, loss_s_tot


# speedup vs baseline: 1.1500x; 1.1500x over previous
import functools

import jax
import jax.numpy as jnp
from jax import lax
from jax.experimental import pallas as pl
from jax.experimental.pallas import tpu as pltpu
from jax.experimental.pallas import tpu_sc as plsc

N_U = 30000
N_I = 20000
E = 800000
D = 64
L = 2
B = 4096
N = N_U + N_I
ALPHA = 0.2
INV_TEMP = 2.0
LAM1 = 0.2
LAM2 = 0.2
REG = 1e-5

_SC_TILES = 16
_STREAM = 128
_SCHUNK = 4
_HD = D // 2
_TSTREAMS = -(-E // (_SC_TILES * _STREAM))
_TSTREAMS = -(-_TSTREAMS // _SCHUNK) * _SCHUNK
_EP = _SC_TILES * _TSTREAMS * _STREAM
_NSC = _TSTREAMS // _SCHUNK
_NSUPT = _SC_TILES * _NSC
_ACC_ROWS = 32768
_DUMMY = N_U
_ZROWS = 128


def _sc_mesh():
    return plsc.VectorSubcoreMesh(core_axis_name="c", subcore_axis_name="s")


_BCAST_DN = lax.GatherDimensionNumbers(
    offset_dims=(), collapsed_slice_dims=(0,), start_index_map=(0,))


def _bcast16(v16, t):
    idx = jnp.full((16, 1), t, jnp.int32)
    return lax.gather(v16, idx, _BCAST_DN, (1,),
                      mode=lax.GatherScatterMode.PROMISE_IN_BOUNDS)


def _spmm_body(scaled, n_out, table, pack, out, idx_v, bufs, zbuf, acc, gsem,
               ssem):
    c = lax.axis_index("c")
    s = lax.axis_index("s")

    for r in range(_ZROWS):
        for q in range(_HD // 16):
            zbuf[r, pl.ds(q * 16, 16)] = jnp.zeros((16,), jnp.float32)
    rows_per_tile = _ACC_ROWS // _SC_TILES
    for t in range(rows_per_tile // _ZROWS):
        pltpu.sync_copy(zbuf,
                        acc.at[pl.ds(s * rows_per_tile + t * _ZROWS, _ZROWS)])
    plsc.subcore_barrier()

    def fire(k, par):
        sup = s * _NSC + k
        pltpu.sync_copy(pack.at[c, sup], idx_v.at[par])
        for j in range(_SCHUNK):
            pltpu.async_copy(table.at[idx_v.at[par, 0, j]],
                             bufs.at[par, j], gsem.at[par])

    def process(par):
        for j in range(_SCHUNK):
            pltpu.make_async_copy(table.at[idx_v.at[par, 0, j]],
                                  bufs.at[par, j], gsem.at[par]).wait()
            if scaled:
                for g16 in range(_STREAM // 16):
                    v16 = plsc.bitcast(
                        idx_v[par, 2, j, pl.ds(g16 * 16, 16)], jnp.float32)
                    for t in range(16):
                        r = g16 * 16 + t
                        bc = _bcast16(v16, t)
                        for q in range(_HD // 16):
                            bufs[par, j, r, pl.ds(q * 16, 16)] = (
                                bufs[par, j, r, pl.ds(q * 16, 16)] * bc)
            pltpu.async_copy(bufs.at[par, j], acc.at[idx_v.at[par, 1, j]],
                             ssem.at[par], add=True)

    def drain_scatter(par):
        for j in range(_SCHUNK):
            pltpu.make_async_copy(bufs.at[par, j],
                                  acc.at[idx_v.at[par, 1, j]],
                                  ssem.at[par]).wait()

    fire(0, 0)

    def step(k, carry):
        par = k % 2

        @pl.when(k >= 1)
        def _():
            drain_scatter(1 - par)

        @pl.when(k + 1 < _NSC)
        def _():
            fire(k + 1, 1 - par)

        process(par)
        return carry

    lax.fori_loop(0, _NSC, step, 0, unroll=False)
    drain_scatter((_NSC - 1) % 2)
    plsc.subcore_barrier()

    rows_per = (n_out // _SC_TILES) // 8 * 8
    tail = n_out - 15 * rows_per

    @pl.when(s < 15)
    def _():
        o = pl.multiple_of(s * rows_per, 8)
        pltpu.sync_copy(acc.at[pl.ds(o, rows_per)],
                        out.at[c, pl.ds(o, rows_per), :])

    @pl.when(s == 15)
    def _():
        pltpu.sync_copy(acc.at[pl.ds(15 * rows_per, tail)],
                        out.at[c, pl.ds(15 * rows_per, tail), :])


def _make_spmm(scaled, n_out):
    body = functools.partial(_spmm_body, scaled, n_out)
    kern = pl.kernel(
        body,
        out_type=jax.ShapeDtypeStruct((2, n_out, _HD), jnp.float32),
        mesh=_sc_mesh(),
        scratch_types=[
            pltpu.VMEM((2, 3, _SCHUNK, _STREAM), jnp.int32),
            pltpu.VMEM((2, _SCHUNK, _STREAM, _HD), jnp.float32),
            pltpu.VMEM((_ZROWS, _HD), jnp.float32),
            pltpu.VMEM_SHARED((_ACC_ROWS, _HD), jnp.float32),
            pltpu.SemaphoreType.DMA((2,)),
            pltpu.SemaphoreType.DMA((2,)),
        ],
        compiler_params=pltpu.CompilerParams(use_tc_tiling_on_sc=False,
                                             needs_layout_passes=False),
    )
    return kern


_GTASKS = (
    (0, 0), (0, 1), (3, 0), (3, 1),
    (1, 0), (1, 1), (4, 0), (4, 1),
    (2, 0), (2, 1), (5, 0), (5, 1),
    (6, 0), (6, 1), (6, 2),
)
_NG = len(_GTASKS)


def _gather_body(e3_0, e1_0, e2_0, e3_1, e1_1, e2_1, latsf, idx3, rawidx,
                 g_out, mask_out, idx_v, buf, idxfull, tbl, mbuf, sem):
    c = lax.axis_index("c")
    s = lax.axis_index("s")
    w = s * 2 + c
    tables = (e3_0, e1_0, e2_0, e3_1, e1_1, e2_1, latsf)

    for p in range(3):
        pltpu.sync_copy(idx3.at[pl.ds(p * B + w * _STREAM, _STREAM)],
                        idx_v.at[p])

    def desc(t):
        tb, p = _GTASKS[t]
        return pltpu.make_async_copy(tables[tb].at[idx_v.at[p]], buf.at[t % 4],
                                     sem)

    for t in range(_NG):
        desc(t).start()
        if t >= 3:
            desc(t - 3).wait()
            pltpu.sync_copy(buf.at[(t - 3) % 4],
                            g_out.at[t - 3, pl.ds(w * _STREAM, _STREAM), :])
    for t in range(_NG - 3, _NG):
        desc(t).wait()
        pltpu.sync_copy(buf.at[t % 4],
                        g_out.at[t, pl.ds(w * _STREAM, _STREAM), :])

    @pl.when(s == 0)
    def _():
        co = pl.multiple_of(c * B, 8)
        pltpu.sync_copy(rawidx.at[pl.ds(co, B)], idxfull)
        for gq in range(B // 16):
            i16 = idxfull[pl.ds(gq * 16, 16)]
            p16 = lax.iota(jnp.int32, 16) + jnp.full((16,), gq * 16, jnp.int32)
            plsc.store_scatter(tbl, [i16], p16)
        for gq in range(B // 16):
            i16 = idxfull[pl.ds(gq * 16, 16)]
            p16 = lax.iota(jnp.int32, 16) + jnp.full((16,), gq * 16, jnp.int32)
            rep = plsc.load_gather(tbl, [i16])
            mbuf[pl.ds(gq * 16, 16)] = (rep == p16).astype(jnp.float32)
        pltpu.sync_copy(mbuf, mask_out.at[pl.ds(co, B)])


def _make_gather():
    return pl.kernel(
        _gather_body,
        out_type=[jax.ShapeDtypeStruct((_NG, B, D), jnp.float32),
                  jax.ShapeDtypeStruct((2 * B,), jnp.float32)],
        mesh=_sc_mesh(),
        scratch_types=[
            pltpu.VMEM((3, _STREAM), jnp.int32),
            pltpu.VMEM((4, _STREAM, D), jnp.float32),
            pltpu.VMEM((B,), jnp.int32),
            pltpu.VMEM((N_U,), jnp.int32),
            pltpu.VMEM((B,), jnp.float32),
            pltpu.SemaphoreType.DMA,
        ],
        compiler_params=pltpu.CompilerParams(use_tc_tiling_on_sc=False,
                                             needs_layout_passes=False),
    )


_BLK = 2000
_NBLK = N // _BLK
_NBLK_U = N_U // _BLK


def _stats0_kernel(e_ref, h_ref, out_ref):
    i = pl.program_id(0)

    @pl.when(i == 0)
    def _():
        out_ref[...] = jnp.zeros_like(out_ref)

    e = e_ref[...]
    h = h_ref[...]
    out_ref[0, :] += jnp.sum(e, axis=0)
    out_ref[1, :] += jnp.sum(e * e, axis=0)
    out_ref[2, :] += jnp.sum(h * h, axis=0)


def _tc_stats0(embeds, hyp):
    return pl.pallas_call(
        _stats0_kernel,
        grid=(_NBLK,),
        in_specs=[pl.BlockSpec((_BLK, D), lambda i: (i, 0)),
                  pl.BlockSpec((_BLK, D), lambda i: (i, 0))],
        out_specs=pl.BlockSpec((8, D), lambda i: (0, 0)),
        out_shape=jax.ShapeDtypeStruct((8, D), jnp.float32),
        compiler_params=pltpu.CompilerParams(dimension_semantics=("arbitrary",)),
    )(embeds, hyp)


def _bn_kernel(lats_ref, huu_ref, par_ref, bn_ref, g_ref):
    i = pl.program_id(0)
    bn = lats_ref[...] * par_ref[0, :] + par_ref[1, :]
    bn_ref[0] = bn[:, :_HD]
    bn_ref[1] = bn[:, _HD:]

    @pl.when((i == 0) | (i == _NBLK_U))
    def _():
        g_ref[...] = jnp.zeros_like(g_ref)

    g_ref[0] += lax.dot_general(huu_ref[...], bn, (((0,), (0,)), ((), ())),
                                preferred_element_type=jnp.float32)


def _tc_bn(lats, huu, par):
    return pl.pallas_call(
        _bn_kernel,
        grid=(_NBLK,),
        in_specs=[pl.BlockSpec((_BLK, D), lambda i: (i, 0)),
                  pl.BlockSpec((_BLK, D), lambda i: (i, 0)),
                  pl.BlockSpec((8, D), lambda i: (0, 0))],
        out_specs=[pl.BlockSpec((2, _BLK, _HD), lambda i: (0, i, 0)),
                   pl.BlockSpec((1, D, D),
                                lambda i: (jnp.where(i < _NBLK_U, 0, 1), 0, 0))],
        out_shape=[jax.ShapeDtypeStruct((2, N, _HD), jnp.float32),
                   jax.ShapeDtypeStruct((2, D, D), jnp.float32)],
        compiler_params=pltpu.CompilerParams(dimension_semantics=("arbitrary",)),
    )(lats, huu, par)


def _combine_kernel(lats_ref, e1_ref, huu_ref, g_ref, e2_ref, e3_ref, nl_ref,
                    st_ref):
    i = pl.program_id(0)
    e2 = lax.dot_general(huu_ref[...], g_ref[0], (((1,), (0,)), ((), ())),
                         preferred_element_type=jnp.float32)
    e3 = e1_ref[...] + lats_ref[...]
    nrm = jnp.sqrt(jnp.sum(e2 * e2, axis=1, keepdims=True))
    nl = e3 + ALPHA * e2 / jnp.maximum(nrm, 1e-12)
    e2_ref[...] = e2
    e3_ref[...] = e3
    nl_ref[...] = nl

    @pl.when(i == 0)
    def _():
        st_ref[...] = jnp.zeros_like(st_ref)

    st_ref[0, :] += jnp.sum(nl, axis=0)
    st_ref[1, :] += jnp.sum(nl * nl, axis=0)


def _tc_combine(lats, e1, huu, G):
    return pl.pallas_call(
        _combine_kernel,
        grid=(_NBLK,),
        in_specs=[pl.BlockSpec((_BLK, D), lambda i: (i, 0)),
                  pl.BlockSpec((_BLK, D), lambda i: (i, 0)),
                  pl.BlockSpec((_BLK, D), lambda i: (i, 0)),
                  pl.BlockSpec((1, D, D),
                               lambda i: (jnp.where(i < _NBLK_U, 0, 1), 0, 0))],
        out_specs=[pl.BlockSpec((_BLK, D), lambda i: (i, 0)),
                   pl.BlockSpec((_BLK, D), lambda i: (i, 0)),
                   pl.BlockSpec((_BLK, D), lambda i: (i, 0)),
                   pl.BlockSpec((8, D), lambda i: (0, 0))],
        out_shape=[jax.ShapeDtypeStruct((N, D), jnp.float32),
                   jax.ShapeDtypeStruct((N, D), jnp.float32),
                   jax.ShapeDtypeStruct((N, D), jnp.float32),
                   jax.ShapeDtypeStruct((8, D), jnp.float32)],
        compiler_params=pltpu.CompilerParams(dimension_semantics=("arbitrary",)),
    )(lats, e1, huu, G)


_RB = 256
_NRB = B // _RB
_P2SLOT = [4, 5, 6, 7, 8, 9, 10, 11]


def _contrast_kernel(p1_ref, p2_ref, p2d_ref, mc_ref, mr_ref, out_ref):
    rb = pl.program_id(1)
    p2 = p2_ref[0]
    p2 = p2 / jnp.maximum(
        jnp.sqrt(jnp.sum(p2 * p2, axis=1, keepdims=True)), 1e-12)
    p1 = p1_ref[0]
    p1 = p1 / jnp.maximum(
        jnp.sqrt(jnp.sum(p1 * p1, axis=1, keepdims=True)), 1e-12)
    p1 = p1 * INV_TEMP
    s = lax.dot_general(p1.astype(jnp.bfloat16), p2.astype(jnp.bfloat16),
                        (((1,), (1,)), ((), ())),
                        preferred_element_type=jnp.float32)
    mc = mc_ref[0, 0, :]
    ex = jnp.exp(s) * mc[None, :]
    lse = jnp.log(jnp.sum(ex, axis=1))
    p2d = p2d_ref[0]
    p2d = p2d / jnp.maximum(
        jnp.sqrt(jnp.sum(p2d * p2d, axis=1, keepdims=True)), 1e-12)
    diag = jnp.sum(p1 * p2d, axis=1)
    mr = mr_ref[0, 0, :]
    contrib = jnp.where(mr > 0.0, diag - lse, 0.0)
    part = jnp.sum(contrib.reshape(_RB // 128, 128), axis=0)

    @pl.when(rb == 0)
    def _():
        out_ref[...] = jnp.zeros_like(out_ref)

    out_ref[0, 0, :] += part


def _tc_contrast(g, masks):
    return pl.pallas_call(
        _contrast_kernel,
        grid=(8, _NRB),
        in_specs=[
            pl.BlockSpec((1, _RB, D), lambda c, rb: (c, rb, 0)),
            pl.BlockSpec((1, B, D), lambda c, rb: (c + 4, 0, 0)),
            pl.BlockSpec((1, _RB, D), lambda c, rb: (c + 4, rb, 0)),
            pl.BlockSpec((1, 1, B), lambda c, rb: (c % 2, 0, 0)),
            pl.BlockSpec((1, 1, _RB), lambda c, rb: (c % 2, 0, rb)),
        ],
        out_specs=pl.BlockSpec((1, 8, 128), lambda c, rb: (c, 0, 0)),
        out_shape=jax.ShapeDtypeStruct((8, 8, 128), jnp.float32),
        compiler_params=pltpu.CompilerParams(
            dimension_semantics=("arbitrary", "arbitrary")),
    )(g, g, g, masks, masks)


def _bpr_kernel(u_ref, p_ref, n_ref, out_ref):
    u = u_ref[0]
    ps = jnp.sum(u * p_ref[0], axis=1)
    ns = jnp.sum(u * n_ref[0], axis=1)
    x = (ps - ns) * (1.0 / D)
    prob = 1.0 / (1.0 + jnp.exp(-x))
    lg = jnp.log(prob + 1e-15)
    part = jnp.sum(lg.reshape(B // 128, 128), axis=0)
    row = lax.broadcasted_iota(jnp.int32, (8, 128), 0)
    out_ref[...] = jnp.where(row == 0, part[None, :], 0.0)


def _tc_bpr(g):
    return pl.pallas_call(
        _bpr_kernel,
        grid=(1,),
        in_specs=[pl.BlockSpec((1, B, D), lambda i: (12, 0, 0)),
                  pl.BlockSpec((1, B, D), lambda i: (13, 0, 0)),
                  pl.BlockSpec((1, B, D), lambda i: (14, 0, 0))],
        out_specs=pl.BlockSpec((8, 128), lambda i: (0, 0)),
        out_shape=jax.ShapeDtypeStruct((8, 128), jnp.float32),
    )(g, g, g)


def _bn_params(colsum, colsumsq, gamma, beta):
    m = colsum / N
    v = colsumsq / N - m * m
    scale = gamma * lax.rsqrt(v + 1e-5)
    shift = beta - m * scale
    return jnp.concatenate(
        [scale[None, :], shift[None, :], jnp.zeros((6, D), jnp.float32)], axis=0)


def kernel(uids, iids, pos, neg, adj_rows, adj_cols, adj_vals, adj_norm_vals,
           u_embeds, i_embeds, u_hyper_graph, i_hyper_graph, bn_gamma, bn_beta):
    del iids, adj_vals
    embeds = jnp.concatenate([u_embeds, i_embeds], axis=0)
    hyp = jnp.concatenate([u_hyper_graph, i_hyper_graph], axis=0)

    pad = _EP - E
    shp = (_NSUPT, _SCHUNK, _STREAM)
    scat_u = jnp.pad(adj_rows, (0, pad), constant_values=_DUMMY).reshape(shp)
    scat_i = jnp.pad(adj_cols, (0, pad), constant_values=_DUMMY).reshape(shp)
    gcols_p = jnp.pad(adj_cols + N_U, (0, pad), constant_values=0).reshape(shp)
    grows_p = jnp.pad(adj_rows, (0, pad), constant_values=0).reshape(shp)
    vals_p = lax.bitcast_convert_type(
        jnp.pad(adj_norm_vals, (0, pad)).reshape(shp), jnp.int32)

    def mkpack(gp, sp):
        return jnp.stack([jnp.stack([gp, sp, vals_p], axis=1),
                          jnp.stack([gp + N, sp, vals_p], axis=1)])

    pack_u = mkpack(gcols_p, scat_u)
    pack_i = mkpack(grows_p, scat_i)

    spmm_u = _make_spmm(False, N_U)
    spmm_i = _make_spmm(False, N_I)
    spmm_us = _make_spmm(True, N_U)
    spmm_is = _make_spmm(True, N_I)

    def split2(x):
        return jnp.concatenate([x[:, :_HD], x[:, _HD:]], axis=0)

    def join2(u2, i2):
        return jnp.concatenate(
            [jnp.concatenate([u2[0], u2[1]], axis=1),
             jnp.concatenate([i2[0], i2[1]], axis=1)], axis=0)

    hyp2 = split2(hyp)
    huu = join2(spmm_u(hyp2, pack_u), spmm_i(hyp2, pack_i))
    stats0 = _tc_stats0(embeds, hyp)

    lats = embeds
    colsum, colsumsq = stats0[0], stats0[1]
    e1s, e2s, e3s = [], [], []
    for l in range(L):
        par = _bn_params(colsum, colsumsq, bn_gamma[l], bn_beta[l])
        bn2, G = _tc_bn(lats, huu, par)
        bn2f = bn2.reshape(2 * N, _HD)
        e1 = join2(spmm_us(bn2f, pack_u), spmm_is(bn2f, pack_i))
        e2, e3, nlats, st = _tc_combine(lats, e1, huu, G)
        e1s.append(e1)
        e2s.append(e2)
        e3s.append(e3)
        lats = nlats
        colsum, colsumsq = st[0], st[1]

    idx3 = jnp.concatenate([uids, pos + N_U, neg + N_U])
    rawidx = jnp.concatenate([uids, pos])
    g, masks = _make_gather()(e3s[0], e1s[0], e2s[0], e3s[1], e1s[1], e2s[1],
                              lats, idx3, rawidx)

    lsum = _tc_contrast(g, masks.reshape(2, 1, B))
    bpr = _tc_bpr(g)

    cnt = jnp.sum(masks.reshape(2, B), axis=1)
    d_c = jnp.sum(lsum, axis=(1, 2))
    loss_c = -d_c / jnp.where(jnp.arange(8) % 2 == 0, cnt[0], cnt[1])
    loss_s = jnp.sum(loss_c[0:4])
    loss_s2 = jnp.sum(loss_c[4:8])
    loss_s_tot = (loss_s * LAM1 + loss_s2 * LAM2) / L

    loss_r = -jnp.sum(bpr) / B

    loss_reg = REG * (jnp.sum(stats0[1]) + jnp.sum(stats0[2])
                      + jnp.sum(bn_gamma ** 2) + jnp.sum(bn_beta ** 2))

    loss = loss_r + loss_s_tot + loss_reg
    return loss, loss_r, loss_s_tot
